# Initial kernel scaffold; baseline (speedup 1.0000x reference)
import jax
import jax.numpy as jnp
from jax.experimental import pallas as pl


def kernel(query, keys_mem, values_mem, W_dg, W_ca1):
    raise NotImplementedError("write your pallas kernel here")


# speedup vs baseline: 9.9279x; 9.9279x over previous
import functools

import jax
import jax.numpy as jnp
from jax import lax
from jax.experimental import pallas as pl
from jax.experimental.pallas import tpu as pltpu
from jax.experimental.pallas import tpu_sc as plsc

B = 1024
D = 64
CAP = 100000
K = 32
CBLK = 1024
CPAD = 100352
NTILES = CPAD // CBLK
EPS = 1e-8
NEG = float("-inf")
IMAX = 2**31 - 1


def _topk_body(q_ref, wdg_ref, keys_ref, blk_ref, half_ref, nov_ref,
               keyn_s, topv_s, topi_s, sim_s):
    c = pl.program_id(0)

    @pl.when(c == 0)
    def _init():
        x = lax.dot_general(q_ref[...], wdg_ref[...],
                            (((1,), (1,)), ((), ())),
                            preferred_element_type=jnp.float32)
        ii = lax.broadcasted_iota(jnp.int32, (B, D), 1)
        rank = jnp.zeros((B, D), jnp.int32)
        for j in range(D):
            xj = x[:, j:j + 1]
            rank += ((xj > x) | ((xj == x) & (ii > j))).astype(jnp.int32)
        xm = jnp.where(rank < K, x, 0.0)
        n2 = jnp.sum(xm * xm, axis=1, keepdims=True)
        keyn_s[...] = xm / jnp.maximum(jnp.sqrt(n2), EPS)
        topv_s[...] = jnp.full((B, K), NEG, jnp.float32)
        topi_s[...] = jnp.zeros((B, K), jnp.int32)

    kt = keys_ref[...]
    n2 = jnp.sum(kt * kt, axis=1, keepdims=True)
    ktn = kt / jnp.maximum(jnp.sqrt(n2), EPS)
    sim = lax.dot_general(keyn_s[...], ktn,
                          (((1,), (1,)), ((), ())),
                          preferred_element_type=jnp.float32)
    gidx = c * CBLK + lax.broadcasted_iota(jnp.int32, (B, CBLK), 1)
    sim_s[...] = jnp.where(gidx < CAP, sim, NEG)

    lane = lax.broadcasted_iota(jnp.int32, (B, K), 1)

    def body(j, carry):
        ov, oi, nv, ni = carry
        xv = sim_s[...]
        ms = jnp.max(xv, axis=1, keepdims=True)
        mr = jnp.max(ov, axis=1, keepdims=True)
        m = jnp.maximum(ms, mr)
        sel_s = jnp.min(jnp.where(xv == m, gidx, IMAX), axis=1,
                        keepdims=True)
        sel_r = jnp.min(jnp.where(ov == m, oi, IMAX), axis=1, keepdims=True)
        sel = jnp.minimum(sel_s, sel_r)
        sim_s[...] = jnp.where(gidx == sel, NEG, xv)
        ov = jnp.where(oi == sel, NEG, ov)
        nv = jnp.where(lane == j, m, nv)
        ni = jnp.where(lane == j, sel, ni)
        return ov, oi, nv, ni

    _, _, nv, ni = lax.fori_loop(
        0, K, body,
        (topv_s[...], topi_s[...],
         jnp.zeros((B, K), jnp.float32), jnp.zeros((B, K), jnp.int32)))
    topv_s[...] = nv
    topi_s[...] = ni

    @pl.when(c == NTILES - 1)
    def _fin():
        ti = topi_s[...]
        blk_ref[...] = ti >> 1
        half_ref[...] = ti & 1
        tv0 = topv_s[...][:, 0:1]
        nov_ref[...] = jnp.clip(1.0 - jnp.clip(tv0, -1.0, 1.0), 0.0, 1.0)


def _topk_call(query, keys_pad, w_dg):
    return pl.pallas_call(
        _topk_body,
        grid=(NTILES,),
        in_specs=[
            pl.BlockSpec((B, D), lambda c: (0, 0)),
            pl.BlockSpec((D, D), lambda c: (0, 0)),
            pl.BlockSpec((CBLK, D), lambda c: (c, 0)),
        ],
        out_specs=[
            pl.BlockSpec((B, K), lambda c: (0, 0)),
            pl.BlockSpec((B, K), lambda c: (0, 0)),
            pl.BlockSpec((B, 1), lambda c: (0, 0)),
        ],
        out_shape=[
            jax.ShapeDtypeStruct((B, K), jnp.int32),
            jax.ShapeDtypeStruct((B, K), jnp.int32),
            jax.ShapeDtypeStruct((B, 1), jnp.float32),
        ],
        scratch_shapes=[
            pltpu.VMEM((B, D), jnp.float32),
            pltpu.VMEM((B, K), jnp.float32),
            pltpu.VMEM((B, K), jnp.int32),
            pltpu.VMEM((B, CBLK), jnp.float32),
        ],
        compiler_params=pltpu.CompilerParams(
            dimension_semantics=("arbitrary",)),
    )(query, w_dg, keys_pad)


_NC = 2
_NS = 16
_NW = _NC * _NS
_BPW = (B * K) // _NW
_CHUNK = 128
_NCHUNK = _BPW // _CHUNK


def _gather_call(blk_idx, table128):
    mesh = plsc.VectorSubcoreMesh(core_axis_name="c", subcore_axis_name="s")

    @functools.partial(
        pl.kernel,
        mesh=mesh,
        out_type=jax.ShapeDtypeStruct((B * K, 2 * D), jnp.float32),
        scratch_types=[
            pltpu.VMEM((_NCHUNK, _CHUNK), jnp.int32),
            pltpu.VMEM((2, _CHUNK, 2 * D), jnp.float32),
            pltpu.SemaphoreType.DMA,
        ],
    )
    def _gather(idx_hbm, table_hbm, out_hbm, idx_v, bufs, sem):
        wid = lax.axis_index("s") * _NC + lax.axis_index("c")
        base = wid * _BPW
        pltpu.sync_copy(idx_hbm.at[pl.ds(wid * _NCHUNK, _NCHUNK)], idx_v)
        handles = []
        for j in range(_NCHUNK):
            handles.append(
                pltpu.async_copy(table_hbm.at[idx_v.at[j]],
                                 bufs.at[j % 2], sem))
            if j > 0:
                handles[j - 1].wait()
                pltpu.sync_copy(
                    bufs.at[(j - 1) % 2],
                    out_hbm.at[pl.ds(base + (j - 1) * _CHUNK, _CHUNK)])
        handles[-1].wait()
        pltpu.sync_copy(
            bufs.at[(_NCHUNK - 1) % 2],
            out_hbm.at[pl.ds(base + (_NCHUNK - 1) * _CHUNK, _CHUNK)])

    return _gather(blk_idx, table128)


def _ca1_body(g_ref, h_ref, w_ref, o_ref):
    g = g_ref[...]
    h = h_ref[...]
    lo = g[:, :D]
    hi = g[:, D:]
    rows = jnp.where(h == 0, lo, hi)
    o_ref[...] = lax.dot_general(rows, w_ref[...],
                                 (((1,), (1,)), ((), ())),
                                 preferred_element_type=jnp.float32)


def _ca1_call(g, half, w_ca1):
    rows = B * K
    blk = 8192
    return pl.pallas_call(
        _ca1_body,
        grid=(rows // blk,),
        in_specs=[
            pl.BlockSpec((blk, 2 * D), lambda i: (i, 0)),
            pl.BlockSpec((blk, 1), lambda i: (i, 0)),
            pl.BlockSpec((D, D), lambda i: (0, 0)),
        ],
        out_specs=pl.BlockSpec((blk, D), lambda i: (i, 0)),
        out_shape=jax.ShapeDtypeStruct((rows, D), jnp.float32),
    )(g, half, w_ca1)


def kernel(query, keys_mem, values_mem, W_dg, W_ca1):
    keys_pad = jnp.pad(keys_mem, ((0, CPAD - CAP), (0, 0)))
    blk, half, nov = _topk_call(query, keys_pad, W_dg)
    table128 = values_mem.reshape(CAP // 2, 2 * D)
    gathered = _gather_call(blk.reshape(-1, _CHUNK), table128)
    recalled = _ca1_call(gathered, half.reshape(-1, 1), W_ca1)
    return recalled.reshape(B, K, D), nov.reshape(B)
